# baseline (speedup 1.0000x reference)
import functools

import jax
import jax.numpy as jnp
from jax import lax
from jax.experimental import pallas as pl
from jax.experimental.pallas import tpu as pltpu
from jax.experimental.pallas import tpu_sc as plsc

N = 10000
E = 320000
FIN = 128
NH = 8
FOUT = 64

LN = 16
NSUB = 16
NCORE = 2
NP = 10240
GSZ = 128
G = E // GSZ
NROW = NP // NSUB
BN = 640
CH = 2
NCHUNK = G // CH
EROW = E // 8

_NEG_SLOPE = 0.2

_BCAST_DNUMS = lax.GatherDimensionNumbers(
    offset_dims=(), collapsed_slice_dims=(0,), start_index_map=(0,))


def _vbcast(vec16, lane):
    idx = lax.broadcast(lane, (LN, 1))
    return lax.gather(vec16, idx, _BCAST_DNUMS, (1,),
                      mode=lax.GatherScatterMode.PROMISE_IN_BOUNDS)


def _tc_proj_body(src_ref, trg_ref, ws_ref, wt_ref, asrc_ref, atrg_ref,
                  spt_ref, ss_ref, st_ref, m_ref, msc):
    i = pl.program_id(0)
    nsteps = pl.num_programs(0)
    x = src_ref[...]
    t = trg_ref[...]
    ws = ws_ref[...]
    wt = wt_ref[...]

    for h in range(NH):
        wh = ws[h * FOUT:(h + 1) * FOUT, :]
        spt_ref[h] = lax.dot_general(
            x, wh, (((1,), (1,)), ((), ())),
            preferred_element_type=jnp.float32)

    vs = (ws * asrc_ref[...]).reshape(NH, FOUT, FIN).sum(axis=1)
    vt = (wt * atrg_ref[...]).reshape(NH, FOUT, FIN).sum(axis=1)
    vs2 = jnp.concatenate([vs, vs], axis=0)
    vt2 = jnp.concatenate([vt, vt], axis=0)
    ssb = lax.dot_general(x, vs2, (((1,), (1,)), ((), ())),
                          preferred_element_type=jnp.float32)
    stb = lax.dot_general(t, vt2, (((1,), (1,)), ((), ())),
                          preferred_element_type=jnp.float32)
    ss_ref[...] = ssb
    st_ref[...] = stb

    @pl.when(i == 0)
    def _():
        msc[0] = jnp.float32(-3.0e38)
        msc[1] = jnp.float32(-3.0e38)

    msc[0] = jnp.maximum(msc[0], jnp.max(ssb))
    msc[1] = jnp.maximum(msc[1], jnp.max(stb))

    @pl.when(i == nsteps - 1)
    def _():
        ub = msc[0] + msc[1]
        m = jnp.where(ub >= 0, ub, _NEG_SLOPE * ub)
        m_ref[...] = jnp.full((8, 128), m, jnp.float32)


def _tc_proj(src2, trg2, ws, wt, asrc_col, atrg_col):
    return pl.pallas_call(
        _tc_proj_body,
        grid=(NP // BN,),
        in_specs=[
            pl.BlockSpec((BN, FIN), lambda i: (i, 0)),
            pl.BlockSpec((BN, FIN), lambda i: (i, 0)),
            pl.BlockSpec((NH * FOUT, FIN), lambda i: (0, 0)),
            pl.BlockSpec((NH * FOUT, FIN), lambda i: (0, 0)),
            pl.BlockSpec((NH * FOUT, 1), lambda i: (0, 0)),
            pl.BlockSpec((NH * FOUT, 1), lambda i: (0, 0)),
        ],
        out_specs=[
            pl.BlockSpec((NH, BN, FOUT), lambda i: (0, i, 0)),
            pl.BlockSpec((BN, 2 * NH), lambda i: (i, 0)),
            pl.BlockSpec((BN, 2 * NH), lambda i: (i, 0)),
            pl.BlockSpec((8, 128), lambda i: (0, 0)),
        ],
        out_shape=[
            jax.ShapeDtypeStruct((NH, NP, FOUT), jnp.float32),
            jax.ShapeDtypeStruct((NP, 2 * NH), jnp.float32),
            jax.ShapeDtypeStruct((NP, 2 * NH), jnp.float32),
            jax.ShapeDtypeStruct((8, 128), jnp.float32),
        ],
        scratch_shapes=[pltpu.SMEM((2,), jnp.float32)],
    )(src2, trg2, ws, wt, asrc_col, atrg_col)


_MESH = plsc.VectorSubcoreMesh(core_axis_name="c", subcore_axis_name="s")


@functools.partial(
    pl.kernel,
    mesh=_MESH,
    compiler_params=pltpu.CompilerParams(use_tc_tiling_on_sc=False),
    out_type=(
        jax.ShapeDtypeStruct((EROW, 128), jnp.float32),
        jax.ShapeDtypeStruct((NCORE, NP, 2 * NH), jnp.float32),
    ),
    scratch_types=[
        pltpu.VMEM_SHARED((NP, 2 * NH), jnp.float32),
        pltpu.VMEM_SHARED((NP, 2 * NH), jnp.float32),
        pltpu.VMEM_SHARED((NP, 2 * NH), jnp.float32),
        pltpu.VMEM((GSZ,), jnp.int32),
        pltpu.VMEM((GSZ,), jnp.int32),
        pltpu.VMEM((GSZ, 2 * NH), jnp.float32),
        pltpu.VMEM((GSZ, 2 * NH), jnp.float32),
        pltpu.VMEM((GSZ, 2 * NH), jnp.float32),
        pltpu.VMEM((GSZ // 8, 128), jnp.float32),
        pltpu.VMEM((LN,), jnp.float32),
        pltpu.SemaphoreType.DMA,
        pltpu.SemaphoreType.DMA,
    ],
)
def _sc_exp(ss_hbm, st_hbm, sti_hbm, m_hbm, z16_hbm,
            expt_hbm, denp_hbm,
            ss_sh, st_sh, den_sh, siv, tiv, arows, brows, erows, eflat, mbuf,
            sem1, sem2):
    c = lax.axis_index("c")
    s = lax.axis_index("s")
    rlo = s * NROW

    pltpu.sync_copy(ss_hbm.at[pl.ds(rlo, NROW), :], ss_sh.at[pl.ds(rlo, NROW), :])
    pltpu.sync_copy(st_hbm.at[pl.ds(rlo, NROW), :], st_sh.at[pl.ds(rlo, NROW), :])
    pltpu.sync_copy(z16_hbm, den_sh.at[pl.ds(rlo, NROW), :])
    pltpu.sync_copy(m_hbm, mbuf)
    plsc.subcore_barrier()

    mvec = mbuf[...]
    ghalf = G // NCORE
    lo = c * ghalf + ((s * ghalf) >> 4)
    hi = c * ghalf + (((s + 1) * ghalf) >> 4)

    def body(g, _):
        pltpu.sync_copy(sti_hbm.at[g, 0, :], siv)
        pltpu.sync_copy(sti_hbm.at[g, 1, :], tiv)
        cp1 = pltpu.async_copy(ss_sh.at[siv], arows, sem1)
        cp2 = pltpu.async_copy(st_sh.at[tiv], brows, sem2)
        cp1.wait()
        cp2.wait()
        for v in range(GSZ):
            u = arows[v, :] + brows[v, :]
            e16 = jnp.exp(jnp.maximum(u, _NEG_SLOPE * u) - mvec)
            erows[v, :] = e16
            eflat[v >> 3, pl.ds((v & 7) * LN, LN)] = e16
        pltpu.sync_copy(erows, den_sh.at[tiv], add=True)
        pltpu.sync_copy(eflat, expt_hbm.at[pl.ds(g * (GSZ // 8), GSZ // 8), :])
        return ()

    lax.fori_loop(lo, hi, body, (), unroll=False)

    plsc.subcore_barrier()
    pltpu.sync_copy(den_sh.at[pl.ds(rlo, NROW), :],
                    denp_hbm.at[c, pl.ds(rlo, NROW), :])


EVC = GSZ // 8


@functools.partial(
    pl.kernel,
    mesh=_MESH,
    compiler_params=pltpu.CompilerParams(use_tc_tiling_on_sc=False),
    out_type=jax.ShapeDtypeStruct((NH, NP, FOUT), jnp.float32),
    scratch_types=[
        pltpu.VMEM_SHARED((NP, FOUT), jnp.float32),
        pltpu.VMEM_SHARED((NP, FOUT), jnp.float32),
        pltpu.VMEM((2, GSZ), jnp.int32),
        pltpu.VMEM((2, GSZ), jnp.int32),
        pltpu.VMEM((EVC, 128), jnp.float32),
        pltpu.VMEM((EVC, 128), jnp.float32),
        pltpu.VMEM((GSZ, FOUT), jnp.float32),
        pltpu.VMEM((GSZ, FOUT), jnp.float32),
        pltpu.SemaphoreType.DMA,
        pltpu.SemaphoreType.DMA,
        pltpu.SemaphoreType.DMA,
        pltpu.SemaphoreType.DMA,
        pltpu.SemaphoreType.DMA,
        pltpu.SemaphoreType.DMA,
    ],
)
def _sc_agg(spt_hbm, expt_hbm, sti_hbm, z64_hbm,
            outt_hbm,
            tab_sh, acc_sh, sti0, sti1, ev0, ev1, rows0, rows1,
            isem0, isem1, gsem0, gsem1, ssem0, ssem1):
    c = lax.axis_index("c")
    s = lax.axis_index("s")
    rlo = s * NROW
    lo = (s * G) >> 4
    hi = ((s + 1) * G) >> 4
    stib = (sti0, sti1)
    evb = (ev0, ev1)
    rowsb = (rows0, rows1)
    isems = (isem0, isem1)
    gsems = (gsem0, gsem1)
    ssems = (ssem0, ssem1)

    def issue_inputs(g, b):
        pltpu.async_copy(sti_hbm.at[g], stib[b], isems[b])
        pltpu.async_copy(expt_hbm.at[pl.ds(g * EVC, EVC), :], evb[b], isems[b])

    def wait_inputs(b):
        pltpu.make_async_copy(sti_hbm.at[0], stib[b], isems[b]).wait()
        pltpu.make_async_copy(expt_hbm.at[pl.ds(0, EVC), :], evb[b],
                              isems[b]).wait()

    def issue_gather(b):
        pltpu.async_copy(tab_sh.at[stib[b].at[0]], rowsb[b], gsems[b])

    def wait_gather(b):
        pltpu.make_async_copy(tab_sh.at[stib[b].at[0]], rowsb[b],
                              gsems[b]).wait()

    def issue_scatter(b):
        pltpu.async_copy(rowsb[b], acc_sh.at[stib[b].at[1]], ssems[b],
                         add=True)

    def wait_scatter(b):
        pltpu.make_async_copy(rowsb[b], acc_sh.at[stib[b].at[1]],
                              ssems[b]).wait()

    for j in range(NH // NCORE):
        h = c * (NH // NCORE) + j
        hidx = lax.broadcast(h, (LN, 1))
        pltpu.sync_copy(spt_hbm.at[h, pl.ds(rlo, NROW), :],
                        tab_sh.at[pl.ds(rlo, NROW), :])
        pltpu.sync_copy(z64_hbm, acc_sh.at[pl.ds(rlo, NROW), :])
        plsc.subcore_barrier()

        issue_inputs(lo, 0)
        wait_inputs(0)
        issue_gather(0)
        issue_inputs(lo + 1, 1)

        def step(g, p, q):
            @pl.when(g > lo)
            def _():
                wait_scatter(q)
            wait_inputs(q)
            issue_gather(q)
            wait_gather(p)
            ev_p = evb[p]
            rows_p = rowsb[p]

            def scale(e8, _):
                for pp in range(8):
                    e = e8 * 8 + pp
                    av = lax.gather(ev_p[e8, pl.ds(pp * LN, LN)], hidx,
                                    _BCAST_DNUMS, (1,),
                                    mode=lax.GatherScatterMode.PROMISE_IN_BOUNDS)
                    for k in range(FOUT // LN):
                        rows_p[e, pl.ds(k * LN, LN)] = (
                            rows_p[e, pl.ds(k * LN, LN)] * av)
                return ()

            lax.fori_loop(0, EVC, scale, (), unroll=4)
            issue_scatter(p)
            issue_inputs(jnp.minimum(g + 2, hi - 1), p)

        def body(g, _):
            par = (g - lo) & 1

            @pl.when(par == 0)
            def _():
                step(g, 0, 1)

            @pl.when(par == 1)
            def _():
                step(g, 1, 0)

            return ()

        lax.fori_loop(lo, hi, body, (), unroll=False)

        t = (hi - lo) & 1

        @pl.when(t == 0)
        def _():
            wait_gather(0)
            wait_scatter(1)
            wait_inputs(1)

        @pl.when(t == 1)
        def _():
            wait_gather(1)
            wait_scatter(0)
            wait_inputs(0)

        plsc.subcore_barrier()
        pltpu.sync_copy(acc_sh.at[pl.ds(rlo, NROW), :],
                        outt_hbm.at[h, pl.ds(rlo, NROW), :])
        plsc.subcore_barrier()


def _tc_merge_body(outt_ref, denp_ref, o_ref):
    d = denp_ref[0] + denp_ref[1]
    r = 1.0 / (d + 1e-16)
    for h in range(NH):
        o_ref[:, h * FOUT:(h + 1) * FOUT] = outt_ref[h] * r[:, h:h + 1]


def _tc_merge(outt, denp):
    return pl.pallas_call(
        _tc_merge_body,
        grid=(NP // BN,),
        in_specs=[
            pl.BlockSpec((NH, BN, FOUT), lambda i: (0, i, 0)),
            pl.BlockSpec((NCORE, BN, 2 * NH), lambda i: (0, i, 0)),
        ],
        out_specs=pl.BlockSpec((BN, NH * FOUT), lambda i: (i, 0)),
        out_shape=jax.ShapeDtypeStruct((NP, NH * FOUT), jnp.float32),
    )(outt, denp)


def kernel(trg, src, edge_index, Wt, Ws, a_trg, a_src):
    src2 = jnp.zeros((NP, FIN), jnp.float32).at[:N].set(src.reshape(N, FIN))
    trg2 = jnp.zeros((NP, FIN), jnp.float32).at[:N].set(trg.reshape(N, FIN))
    asrc_col = a_src.reshape(NH * FOUT, 1)
    atrg_col = a_trg.reshape(NH * FOUT, 1)
    sti = edge_index.astype(jnp.int32).reshape(2, G, GSZ).transpose(1, 0, 2)

    spt, ss, st, m = _tc_proj(src2, trg2, Ws, Wt, asrc_col, atrg_col)
    m16 = m[0, :LN]
    z16 = jnp.zeros((NROW, 2 * NH), jnp.float32)
    z64 = jnp.zeros((NROW, FOUT), jnp.float32)

    expt, denp = _sc_exp(ss, st, sti, m16, z16)
    outt = _sc_agg(spt, expt, sti, z64)
    out = _tc_merge(outt, denp)
    return out[:N].reshape(1, N, NH * FOUT)


# speedup vs baseline: 1.0080x; 1.0080x over previous
import functools

import jax
import jax.numpy as jnp
from jax import lax
from jax.experimental import pallas as pl
from jax.experimental.pallas import tpu as pltpu
from jax.experimental.pallas import tpu_sc as plsc

N = 10000
E = 320000
FIN = 128
NH = 8
FOUT = 64

LN = 16
NSUB = 16
NCORE = 2
NP = 10240
GSZ = 128
G = E // GSZ
NROW = NP // NSUB
BN = 640
CH = 2
NCHUNK = G // CH
EROW = E // 8

_NEG_SLOPE = 0.2

_BCAST_DNUMS = lax.GatherDimensionNumbers(
    offset_dims=(), collapsed_slice_dims=(0,), start_index_map=(0,))


def _vbcast(vec16, lane):
    idx = lax.broadcast(lane, (LN, 1))
    return lax.gather(vec16, idx, _BCAST_DNUMS, (1,),
                      mode=lax.GatherScatterMode.PROMISE_IN_BOUNDS)


def _tc_proj_body(src_ref, trg_ref, ws_ref, wt_ref, asrc_ref, atrg_ref,
                  spt_ref, ss_ref, st_ref, m_ref, msc):
    i = pl.program_id(0)
    nsteps = pl.num_programs(0)
    x = src_ref[...]
    t = trg_ref[...]
    ws = ws_ref[...]
    wt = wt_ref[...]

    for h in range(NH):
        wh = ws[h * FOUT:(h + 1) * FOUT, :]
        spt_ref[h] = lax.dot_general(
            x, wh, (((1,), (1,)), ((), ())),
            preferred_element_type=jnp.float32)

    vs = (ws * asrc_ref[...]).reshape(NH, FOUT, FIN).sum(axis=1)
    vt = (wt * atrg_ref[...]).reshape(NH, FOUT, FIN).sum(axis=1)
    vs2 = jnp.concatenate([vs, vs], axis=0)
    vt2 = jnp.concatenate([vt, vt], axis=0)
    ssb = lax.dot_general(x, vs2, (((1,), (1,)), ((), ())),
                          preferred_element_type=jnp.float32)
    stb = lax.dot_general(t, vt2, (((1,), (1,)), ((), ())),
                          preferred_element_type=jnp.float32)
    ss_ref[...] = ssb
    st_ref[...] = stb

    @pl.when(i == 0)
    def _():
        msc[0] = jnp.float32(-3.0e38)
        msc[1] = jnp.float32(-3.0e38)

    msc[0] = jnp.maximum(msc[0], jnp.max(ssb))
    msc[1] = jnp.maximum(msc[1], jnp.max(stb))

    @pl.when(i == nsteps - 1)
    def _():
        ub = msc[0] + msc[1]
        m = jnp.where(ub >= 0, ub, _NEG_SLOPE * ub)
        m_ref[...] = jnp.full((8, 128), m, jnp.float32)


def _tc_proj(src2, trg2, ws, wt, asrc_col, atrg_col):
    return pl.pallas_call(
        _tc_proj_body,
        grid=(NP // BN,),
        in_specs=[
            pl.BlockSpec((BN, FIN), lambda i: (i, 0)),
            pl.BlockSpec((BN, FIN), lambda i: (i, 0)),
            pl.BlockSpec((NH * FOUT, FIN), lambda i: (0, 0)),
            pl.BlockSpec((NH * FOUT, FIN), lambda i: (0, 0)),
            pl.BlockSpec((NH * FOUT, 1), lambda i: (0, 0)),
            pl.BlockSpec((NH * FOUT, 1), lambda i: (0, 0)),
        ],
        out_specs=[
            pl.BlockSpec((NH, BN, FOUT), lambda i: (0, i, 0)),
            pl.BlockSpec((BN, 2 * NH), lambda i: (i, 0)),
            pl.BlockSpec((BN, 2 * NH), lambda i: (i, 0)),
            pl.BlockSpec((8, 128), lambda i: (0, 0)),
        ],
        out_shape=[
            jax.ShapeDtypeStruct((NH, NP, FOUT), jnp.float32),
            jax.ShapeDtypeStruct((NP, 2 * NH), jnp.float32),
            jax.ShapeDtypeStruct((NP, 2 * NH), jnp.float32),
            jax.ShapeDtypeStruct((8, 128), jnp.float32),
        ],
        scratch_shapes=[pltpu.SMEM((2,), jnp.float32)],
    )(src2, trg2, ws, wt, asrc_col, atrg_col)


_MESH = plsc.VectorSubcoreMesh(core_axis_name="c", subcore_axis_name="s")


@functools.partial(
    pl.kernel,
    mesh=_MESH,
    compiler_params=pltpu.CompilerParams(use_tc_tiling_on_sc=False),
    out_type=(
        jax.ShapeDtypeStruct((EROW, 128), jnp.float32),
        jax.ShapeDtypeStruct((NCORE, NP, 2 * NH), jnp.float32),
    ),
    scratch_types=[
        pltpu.VMEM_SHARED((NP, 2 * NH), jnp.float32),
        pltpu.VMEM_SHARED((NP, 2 * NH), jnp.float32),
        pltpu.VMEM_SHARED((NP, 2 * NH), jnp.float32),
        pltpu.VMEM((GSZ,), jnp.int32),
        pltpu.VMEM((GSZ,), jnp.int32),
        pltpu.VMEM((GSZ, 2 * NH), jnp.float32),
        pltpu.VMEM((GSZ, 2 * NH), jnp.float32),
        pltpu.VMEM((GSZ, 2 * NH), jnp.float32),
        pltpu.VMEM((GSZ // 8, 128), jnp.float32),
        pltpu.VMEM((LN,), jnp.float32),
        pltpu.SemaphoreType.DMA,
        pltpu.SemaphoreType.DMA,
    ],
)
def _sc_exp(ss_hbm, st_hbm, sti_hbm, m_hbm, z16_hbm,
            expt_hbm, denp_hbm,
            ss_sh, st_sh, den_sh, siv, tiv, arows, brows, erows, eflat, mbuf,
            sem1, sem2):
    c = lax.axis_index("c")
    s = lax.axis_index("s")
    rlo = s * NROW

    pltpu.sync_copy(ss_hbm.at[pl.ds(rlo, NROW), :], ss_sh.at[pl.ds(rlo, NROW), :])
    pltpu.sync_copy(st_hbm.at[pl.ds(rlo, NROW), :], st_sh.at[pl.ds(rlo, NROW), :])
    pltpu.sync_copy(z16_hbm, den_sh.at[pl.ds(rlo, NROW), :])
    pltpu.sync_copy(m_hbm, mbuf)
    plsc.subcore_barrier()

    mvec = mbuf[...]
    ghalf = G // NCORE
    lo = c * ghalf + ((s * ghalf) >> 4)
    hi = c * ghalf + (((s + 1) * ghalf) >> 4)

    def body(g, _):
        pltpu.sync_copy(sti_hbm.at[g, 0, :], siv)
        pltpu.sync_copy(sti_hbm.at[g, 1, :], tiv)
        cp1 = pltpu.async_copy(ss_sh.at[siv], arows, sem1)
        cp2 = pltpu.async_copy(st_sh.at[tiv], brows, sem2)
        cp1.wait()
        cp2.wait()
        for v in range(GSZ):
            u = arows[v, :] + brows[v, :]
            e16 = jnp.exp(jnp.maximum(u, _NEG_SLOPE * u) - mvec)
            erows[v, :] = e16
            eflat[v >> 3, pl.ds((v & 7) * LN, LN)] = e16
        pltpu.sync_copy(erows, den_sh.at[tiv], add=True)
        pltpu.sync_copy(eflat, expt_hbm.at[pl.ds(g * (GSZ // 8), GSZ // 8), :])
        return ()

    lax.fori_loop(lo, hi, body, (), unroll=False)

    plsc.subcore_barrier()
    pltpu.sync_copy(den_sh.at[pl.ds(rlo, NROW), :],
                    denp_hbm.at[c, pl.ds(rlo, NROW), :])


EVC = GSZ // 8


@functools.partial(
    pl.kernel,
    mesh=_MESH,
    compiler_params=pltpu.CompilerParams(use_tc_tiling_on_sc=False),
    out_type=jax.ShapeDtypeStruct((NH, NP, FOUT), jnp.float32),
    scratch_types=[
        pltpu.VMEM_SHARED((NP, FOUT), jnp.float32),
        pltpu.VMEM((2, GSZ), jnp.int32),
        pltpu.VMEM((2, GSZ), jnp.int32),
        pltpu.VMEM((GSZ,), jnp.int32),
        pltpu.VMEM((GSZ,), jnp.int32),
        pltpu.VMEM((EVC, 128), jnp.float32),
        pltpu.VMEM((EVC, 128), jnp.float32),
        pltpu.VMEM((GSZ, FOUT), jnp.float32),
        pltpu.VMEM((GSZ, FOUT), jnp.float32),
        pltpu.SemaphoreType.DMA,
        pltpu.SemaphoreType.DMA,
        pltpu.SemaphoreType.DMA,
        pltpu.SemaphoreType.DMA,
        pltpu.SemaphoreType.DMA,
        pltpu.SemaphoreType.DMA,
    ],
)
def _sc_agg(spt_hbm, expt_hbm, sti_hbm, z64_hbm,
            outt_hbm,
            acc_sh, sti0, sti1, sidx0, sidx1, ev0, ev1, rows0, rows1,
            isem0, isem1, gsem0, gsem1, ssem0, ssem1):
    c = lax.axis_index("c")
    s = lax.axis_index("s")
    rlo = s * NROW
    lo = (s * G) >> 4
    hi = ((s + 1) * G) >> 4
    stib = (sti0, sti1)
    sidxb = (sidx0, sidx1)
    evb = (ev0, ev1)
    rowsb = (rows0, rows1)
    isems = (isem0, isem1)
    gsems = (gsem0, gsem1)
    ssems = (ssem0, ssem1)

    def issue_inputs(g, b):
        pltpu.async_copy(sti_hbm.at[g], stib[b], isems[b])
        pltpu.async_copy(expt_hbm.at[pl.ds(g * EVC, EVC), :], evb[b], isems[b])

    def wait_inputs(b):
        pltpu.make_async_copy(sti_hbm.at[0], stib[b], isems[b]).wait()
        pltpu.make_async_copy(expt_hbm.at[pl.ds(0, EVC), :], evb[b],
                              isems[b]).wait()

    def issue_gather(b):
        pltpu.async_copy(spt_hbm.at[sidxb[b]], rowsb[b], gsems[b])

    def wait_gather(b):
        pltpu.make_async_copy(spt_hbm.at[sidxb[b]], rowsb[b],
                              gsems[b]).wait()

    def issue_scatter(b):
        pltpu.async_copy(rowsb[b], acc_sh.at[stib[b].at[1]], ssems[b],
                         add=True)

    def wait_scatter(b):
        pltpu.make_async_copy(rowsb[b], acc_sh.at[stib[b].at[1]],
                              ssems[b]).wait()

    for j in range(NH // NCORE):
        h = c * (NH // NCORE) + j
        hidx = lax.broadcast(h, (LN, 1))
        hoff = lax.broadcast(h * NP, (LN,))

        def shift_idx(b):
            for k in range(GSZ // LN):
                sidxb[b][pl.ds(k * LN, LN)] = (
                    stib[b][0, pl.ds(k * LN, LN)] + hoff)

        pltpu.sync_copy(z64_hbm, acc_sh.at[pl.ds(rlo, NROW), :])
        plsc.subcore_barrier()

        issue_inputs(lo, 0)
        wait_inputs(0)
        shift_idx(0)
        issue_gather(0)
        issue_inputs(lo + 1, 1)

        def step(g, p, q):
            @pl.when(g > lo)
            def _():
                wait_scatter(q)
            wait_inputs(q)
            shift_idx(q)
            issue_gather(q)
            wait_gather(p)
            ev_p = evb[p]
            rows_p = rowsb[p]

            def scale(e8, _):
                for pp in range(8):
                    e = e8 * 8 + pp
                    av = lax.gather(ev_p[e8, pl.ds(pp * LN, LN)], hidx,
                                    _BCAST_DNUMS, (1,),
                                    mode=lax.GatherScatterMode.PROMISE_IN_BOUNDS)
                    for k in range(FOUT // LN):
                        rows_p[e, pl.ds(k * LN, LN)] = (
                            rows_p[e, pl.ds(k * LN, LN)] * av)
                return ()

            lax.fori_loop(0, EVC, scale, (), unroll=4)
            issue_scatter(p)
            issue_inputs(jnp.minimum(g + 2, hi - 1), p)

        def body(g, _):
            par = (g - lo) & 1

            @pl.when(par == 0)
            def _():
                step(g, 0, 1)

            @pl.when(par == 1)
            def _():
                step(g, 1, 0)

            return ()

        lax.fori_loop(lo, hi, body, (), unroll=False)

        t = (hi - lo) & 1

        @pl.when(t == 0)
        def _():
            wait_gather(0)
            wait_scatter(1)
            wait_inputs(1)

        @pl.when(t == 1)
        def _():
            wait_gather(1)
            wait_scatter(0)
            wait_inputs(0)

        plsc.subcore_barrier()
        pltpu.sync_copy(acc_sh.at[pl.ds(rlo, NROW), :],
                        outt_hbm.at[h, pl.ds(rlo, NROW), :])
        plsc.subcore_barrier()


def _tc_merge_body(outt_ref, denp_ref, o_ref):
    d = denp_ref[0] + denp_ref[1]
    r = 1.0 / (d + 1e-16)
    for h in range(NH):
        o_ref[:, h * FOUT:(h + 1) * FOUT] = outt_ref[h] * r[:, h:h + 1]


def _tc_merge(outt, denp):
    return pl.pallas_call(
        _tc_merge_body,
        grid=(NP // BN,),
        in_specs=[
            pl.BlockSpec((NH, BN, FOUT), lambda i: (0, i, 0)),
            pl.BlockSpec((NCORE, BN, 2 * NH), lambda i: (0, i, 0)),
        ],
        out_specs=pl.BlockSpec((BN, NH * FOUT), lambda i: (i, 0)),
        out_shape=jax.ShapeDtypeStruct((NP, NH * FOUT), jnp.float32),
    )(outt, denp)


def kernel(trg, src, edge_index, Wt, Ws, a_trg, a_src):
    src2 = jnp.zeros((NP, FIN), jnp.float32).at[:N].set(src.reshape(N, FIN))
    trg2 = jnp.zeros((NP, FIN), jnp.float32).at[:N].set(trg.reshape(N, FIN))
    asrc_col = a_src.reshape(NH * FOUT, 1)
    atrg_col = a_trg.reshape(NH * FOUT, 1)
    sti = edge_index.astype(jnp.int32).reshape(2, G, GSZ).transpose(1, 0, 2)

    spt, ss, st, m = _tc_proj(src2, trg2, Ws, Wt, asrc_col, atrg_col)
    m16 = m[0, :LN]
    z16 = jnp.zeros((NROW, 2 * NH), jnp.float32)
    z64 = jnp.zeros((NROW, FOUT), jnp.float32)

    expt, denp = _sc_exp(ss, st, sti, m16, z16)
    outt = _sc_agg(spt.reshape(NH * NP, FOUT), expt, sti, z64)
    out = _tc_merge(outt, denp)
    return out[:N].reshape(1, N, NH * FOUT)


# speedup vs baseline: 1.1516x; 1.1424x over previous
import functools

import jax
import jax.numpy as jnp
from jax import lax
from jax.experimental import pallas as pl
from jax.experimental.pallas import tpu as pltpu
from jax.experimental.pallas import tpu_sc as plsc

N = 10000
E = 320000
FIN = 128
NH = 8
FOUT = 64

LN = 16
NSUB = 16
NCORE = 2
NP = 10240
GSZ = 128
G = E // GSZ
NROW = NP // NSUB
BN = 640
CH = 2
NCHUNK = G // CH
EROW = E // 8

_NEG_SLOPE = 0.2

_BCAST_DNUMS = lax.GatherDimensionNumbers(
    offset_dims=(), collapsed_slice_dims=(0,), start_index_map=(0,))


def _vbcast(vec16, lane):
    idx = lax.broadcast(lane, (LN, 1))
    return lax.gather(vec16, idx, _BCAST_DNUMS, (1,),
                      mode=lax.GatherScatterMode.PROMISE_IN_BOUNDS)


def _tc_proj_body(src_ref, trg_ref, ws_ref, wt_ref, asrc_ref, atrg_ref,
                  spt_ref, ss_ref, st_ref, m_ref, msc):
    i = pl.program_id(0)
    nsteps = pl.num_programs(0)
    x = src_ref[...]
    t = trg_ref[...]
    ws = ws_ref[...]
    wt = wt_ref[...]

    for h in range(NH):
        wh = ws[h * FOUT:(h + 1) * FOUT, :]
        spt_ref[h] = lax.dot_general(
            x, wh, (((1,), (1,)), ((), ())),
            preferred_element_type=jnp.float32)

    vs = (ws * asrc_ref[...]).reshape(NH, FOUT, FIN).sum(axis=1)
    vt = (wt * atrg_ref[...]).reshape(NH, FOUT, FIN).sum(axis=1)
    vs2 = jnp.concatenate([vs, vs], axis=0)
    vt2 = jnp.concatenate([vt, vt], axis=0)
    ssb = lax.dot_general(x, vs2, (((1,), (1,)), ((), ())),
                          preferred_element_type=jnp.float32)
    stb = lax.dot_general(t, vt2, (((1,), (1,)), ((), ())),
                          preferred_element_type=jnp.float32)
    ss_ref[...] = ssb
    st_ref[...] = stb

    @pl.when(i == 0)
    def _():
        msc[0] = jnp.float32(-3.0e38)
        msc[1] = jnp.float32(-3.0e38)

    msc[0] = jnp.maximum(msc[0], jnp.max(ssb))
    msc[1] = jnp.maximum(msc[1], jnp.max(stb))

    @pl.when(i == nsteps - 1)
    def _():
        ub = msc[0] + msc[1]
        m = jnp.where(ub >= 0, ub, _NEG_SLOPE * ub)
        m_ref[...] = jnp.full((8, 128), m, jnp.float32)


def _tc_proj(src2, trg2, ws, wt, asrc_col, atrg_col):
    return pl.pallas_call(
        _tc_proj_body,
        grid=(NP // BN,),
        in_specs=[
            pl.BlockSpec((BN, FIN), lambda i: (i, 0)),
            pl.BlockSpec((BN, FIN), lambda i: (i, 0)),
            pl.BlockSpec((NH * FOUT, FIN), lambda i: (0, 0)),
            pl.BlockSpec((NH * FOUT, FIN), lambda i: (0, 0)),
            pl.BlockSpec((NH * FOUT, 1), lambda i: (0, 0)),
            pl.BlockSpec((NH * FOUT, 1), lambda i: (0, 0)),
        ],
        out_specs=[
            pl.BlockSpec((NH, BN, FOUT), lambda i: (0, i, 0)),
            pl.BlockSpec((BN, 2 * NH), lambda i: (i, 0)),
            pl.BlockSpec((BN, 2 * NH), lambda i: (i, 0)),
            pl.BlockSpec((8, 128), lambda i: (0, 0)),
        ],
        out_shape=[
            jax.ShapeDtypeStruct((NH, NP, FOUT), jnp.float32),
            jax.ShapeDtypeStruct((NP, 2 * NH), jnp.float32),
            jax.ShapeDtypeStruct((NP, 2 * NH), jnp.float32),
            jax.ShapeDtypeStruct((8, 128), jnp.float32),
        ],
        scratch_shapes=[pltpu.SMEM((2,), jnp.float32)],
    )(src2, trg2, ws, wt, asrc_col, atrg_col)


_MESH = plsc.VectorSubcoreMesh(core_axis_name="c", subcore_axis_name="s")


@functools.partial(
    pl.kernel,
    mesh=_MESH,
    compiler_params=pltpu.CompilerParams(use_tc_tiling_on_sc=False),
    out_type=(
        jax.ShapeDtypeStruct((EROW, 128), jnp.float32),
        jax.ShapeDtypeStruct((NCORE, NP, 2 * NH), jnp.float32),
    ),
    scratch_types=[
        pltpu.VMEM_SHARED((NP, 2 * NH), jnp.float32),
        pltpu.VMEM_SHARED((NP, 2 * NH), jnp.float32),
        pltpu.VMEM_SHARED((NP, 2 * NH), jnp.float32),
        pltpu.VMEM((GSZ,), jnp.int32),
        pltpu.VMEM((GSZ,), jnp.int32),
        pltpu.VMEM((GSZ, 2 * NH), jnp.float32),
        pltpu.VMEM((GSZ, 2 * NH), jnp.float32),
        pltpu.VMEM((GSZ, 2 * NH), jnp.float32),
        pltpu.VMEM((GSZ // 8, 128), jnp.float32),
        pltpu.VMEM((LN,), jnp.float32),
        pltpu.SemaphoreType.DMA,
        pltpu.SemaphoreType.DMA,
    ],
)
def _sc_exp(ss_hbm, st_hbm, sti_hbm, m_hbm, z16_hbm,
            expt_hbm, denp_hbm,
            ss_sh, st_sh, den_sh, siv, tiv, arows, brows, erows, eflat, mbuf,
            sem1, sem2):
    c = lax.axis_index("c")
    s = lax.axis_index("s")
    rlo = s * NROW

    pltpu.sync_copy(ss_hbm.at[pl.ds(rlo, NROW), :], ss_sh.at[pl.ds(rlo, NROW), :])
    pltpu.sync_copy(st_hbm.at[pl.ds(rlo, NROW), :], st_sh.at[pl.ds(rlo, NROW), :])
    pltpu.sync_copy(z16_hbm, den_sh.at[pl.ds(rlo, NROW), :])
    pltpu.sync_copy(m_hbm, mbuf)
    plsc.subcore_barrier()

    mvec = mbuf[...]
    ghalf = G // NCORE
    lo = c * ghalf + ((s * ghalf) >> 4)
    hi = c * ghalf + (((s + 1) * ghalf) >> 4)

    def body(g, _):
        pltpu.sync_copy(sti_hbm.at[g, 0, :], siv)
        pltpu.sync_copy(sti_hbm.at[g, 1, :], tiv)
        cp1 = pltpu.async_copy(ss_sh.at[siv], arows, sem1)
        cp2 = pltpu.async_copy(st_sh.at[tiv], brows, sem2)
        cp1.wait()
        cp2.wait()
        for v in range(GSZ):
            u = arows[v, :] + brows[v, :]
            e16 = jnp.exp(jnp.maximum(u, _NEG_SLOPE * u) - mvec)
            erows[v, :] = e16
            eflat[v >> 3, pl.ds((v & 7) * LN, LN)] = e16
        pltpu.sync_copy(erows, den_sh.at[tiv], add=True)
        pltpu.sync_copy(eflat, expt_hbm.at[pl.ds(g * (GSZ // 8), GSZ // 8), :])
        return ()

    lax.fori_loop(lo, hi, body, (), unroll=False)

    plsc.subcore_barrier()
    pltpu.sync_copy(den_sh.at[pl.ds(rlo, NROW), :],
                    denp_hbm.at[c, pl.ds(rlo, NROW), :])


CG = 2
CSZ = CG * GSZ
EVC = CSZ // 8
NC3 = G // CG


@functools.partial(
    pl.kernel,
    mesh=_MESH,
    compiler_params=pltpu.CompilerParams(use_tc_tiling_on_sc=False),
    out_type=jax.ShapeDtypeStruct((NH, NP, FOUT), jnp.float32),
    scratch_types=[
        pltpu.VMEM_SHARED((NP, FOUT), jnp.float32),
        pltpu.VMEM((CG, GSZ), jnp.int32),
        pltpu.VMEM((CG, GSZ), jnp.int32),
        pltpu.VMEM((CG, GSZ), jnp.int32),
        pltpu.VMEM((CG, GSZ), jnp.int32),
        pltpu.VMEM((CG, GSZ), jnp.int32),
        pltpu.VMEM((CG, GSZ), jnp.int32),
        pltpu.VMEM((CG, GSZ), jnp.int32),
        pltpu.VMEM((CG, GSZ), jnp.int32),
        pltpu.VMEM((EVC, 128), jnp.float32),
        pltpu.VMEM((EVC, 128), jnp.float32),
        pltpu.VMEM((CSZ, FOUT), jnp.float32),
        pltpu.VMEM((CSZ, FOUT), jnp.float32),
        pltpu.SemaphoreType.DMA,
        pltpu.SemaphoreType.DMA,
        pltpu.SemaphoreType.DMA,
        pltpu.SemaphoreType.DMA,
        pltpu.SemaphoreType.DMA,
        pltpu.SemaphoreType.DMA,
    ],
)
def _sc_agg(spt_hbm, expt_hbm, sti_hbm, z64_hbm,
            outt_hbm,
            acc_sh, src0, src1, trg0, trg1, tsn0, tsn1, sidx0, sidx1,
            ev0, ev1, rows0, rows1,
            isem0, isem1, gsem0, gsem1, ssem0, ssem1):
    c = lax.axis_index("c")
    s = lax.axis_index("s")
    rlo = s * NROW
    lo = (s * NC3) >> 4
    hi = ((s + 1) * NC3) >> 4
    srcb = (src0, src1)
    trgb = (trg0, trg1)
    tsnb = (tsn0, tsn1)
    sidxb = (sidx0, sidx1)
    evb = (ev0, ev1)
    rowsb = (rows0, rows1)
    isems = (isem0, isem1)
    gsems = (gsem0, gsem1)
    ssems = (ssem0, ssem1)

    def issue_inputs(g, b):
        pltpu.async_copy(sti_hbm.at[pl.ds(g * CG, CG), 0, :], srcb[b],
                         isems[b])
        pltpu.async_copy(sti_hbm.at[pl.ds(g * CG, CG), 1, :], trgb[b],
                         isems[b])
        pltpu.async_copy(expt_hbm.at[pl.ds(g * EVC, EVC), :], evb[b], isems[b])

    def wait_inputs(b):
        pltpu.make_async_copy(sti_hbm.at[pl.ds(0, CG), 0, :], srcb[b],
                              isems[b]).wait()
        pltpu.make_async_copy(sti_hbm.at[pl.ds(0, CG), 1, :], trgb[b],
                              isems[b]).wait()
        pltpu.make_async_copy(expt_hbm.at[pl.ds(0, EVC), :], evb[b],
                              isems[b]).wait()

    def issue_gather(b):
        for gg in range(CG):
            pltpu.async_copy(spt_hbm.at[sidxb[b].at[gg]],
                             rowsb[b].at[pl.ds(gg * GSZ, GSZ), :], gsems[b])

    def wait_gather(b):
        for gg in range(CG):
            pltpu.make_async_copy(spt_hbm.at[sidxb[b].at[gg]],
                                  rowsb[b].at[pl.ds(gg * GSZ, GSZ), :],
                                  gsems[b]).wait()

    def issue_scatter(b):
        for gg in range(CG):
            for k in range(GSZ // LN):
                tsnb[b][gg, pl.ds(k * LN, LN)] = trgb[b][gg, pl.ds(k * LN, LN)]
        for gg in range(CG):
            pltpu.async_copy(rowsb[b].at[pl.ds(gg * GSZ, GSZ), :],
                             acc_sh.at[tsnb[b].at[gg]], ssems[b],
                             add=True)

    def wait_scatter(b):
        for gg in range(CG):
            pltpu.make_async_copy(rowsb[b].at[pl.ds(gg * GSZ, GSZ), :],
                                  acc_sh.at[tsnb[b].at[gg]],
                                  ssems[b]).wait()

    for j in range(NH // NCORE):
        h = c * (NH // NCORE) + j
        hidx = lax.broadcast(h, (LN, 1))
        hoff = lax.broadcast(h * NP, (LN,))

        def shift_idx(b):
            for gg in range(CG):
                for k in range(GSZ // LN):
                    sidxb[b][gg, pl.ds(k * LN, LN)] = (
                        srcb[b][gg, pl.ds(k * LN, LN)] + hoff)

        pltpu.sync_copy(z64_hbm, acc_sh.at[pl.ds(rlo, NROW), :])
        plsc.subcore_barrier()

        issue_inputs(lo, 0)
        wait_inputs(0)
        shift_idx(0)
        issue_gather(0)
        issue_inputs(lo + 1, 1)

        def step(g, p, q):
            @pl.when(g > lo)
            def _():
                wait_scatter(q)
            wait_inputs(q)
            shift_idx(q)
            issue_gather(q)
            wait_gather(p)
            ev_p = evb[p]
            rows_p = rowsb[p]

            def scale(e8, _):
                for pp in range(8):
                    e = e8 * 8 + pp
                    av = lax.gather(ev_p[e8, pl.ds(pp * LN, LN)], hidx,
                                    _BCAST_DNUMS, (1,),
                                    mode=lax.GatherScatterMode.PROMISE_IN_BOUNDS)
                    for k in range(FOUT // LN):
                        rows_p[e, pl.ds(k * LN, LN)] = (
                            rows_p[e, pl.ds(k * LN, LN)] * av)
                return ()

            lax.fori_loop(0, EVC, scale, (), unroll=4)
            issue_scatter(p)
            issue_inputs(jnp.minimum(g + 2, hi - 1), p)

        def body(g, _):
            par = (g - lo) & 1

            @pl.when(par == 0)
            def _():
                step(g, 0, 1)

            @pl.when(par == 1)
            def _():
                step(g, 1, 0)

            return ()

        lax.fori_loop(lo, hi, body, (), unroll=False)

        t = (hi - lo) & 1

        @pl.when(t == 0)
        def _():
            wait_gather(0)
            wait_scatter(1)
            wait_inputs(1)

        @pl.when(t == 1)
        def _():
            wait_gather(1)
            wait_scatter(0)
            wait_inputs(0)

        plsc.subcore_barrier()
        pltpu.sync_copy(acc_sh.at[pl.ds(rlo, NROW), :],
                        outt_hbm.at[h, pl.ds(rlo, NROW), :])
        plsc.subcore_barrier()


def _tc_merge_body(outt_ref, denp_ref, o_ref):
    d = denp_ref[0] + denp_ref[1]
    r = 1.0 / (d + 1e-16)
    for h in range(NH):
        o_ref[:, h * FOUT:(h + 1) * FOUT] = outt_ref[h] * r[:, h:h + 1]


def _tc_merge(outt, denp):
    return pl.pallas_call(
        _tc_merge_body,
        grid=(NP // BN,),
        in_specs=[
            pl.BlockSpec((NH, BN, FOUT), lambda i: (0, i, 0)),
            pl.BlockSpec((NCORE, BN, 2 * NH), lambda i: (0, i, 0)),
        ],
        out_specs=pl.BlockSpec((BN, NH * FOUT), lambda i: (i, 0)),
        out_shape=jax.ShapeDtypeStruct((NP, NH * FOUT), jnp.float32),
    )(outt, denp)


def kernel(trg, src, edge_index, Wt, Ws, a_trg, a_src):
    src2 = jnp.zeros((NP, FIN), jnp.float32).at[:N].set(src.reshape(N, FIN))
    trg2 = jnp.zeros((NP, FIN), jnp.float32).at[:N].set(trg.reshape(N, FIN))
    asrc_col = a_src.reshape(NH * FOUT, 1)
    atrg_col = a_trg.reshape(NH * FOUT, 1)
    sti = edge_index.astype(jnp.int32).reshape(2, G, GSZ).transpose(1, 0, 2)

    spt, ss, st, m = _tc_proj(src2, trg2, Ws, Wt, asrc_col, atrg_col)
    m16 = m[0, :LN]
    z16 = jnp.zeros((NROW, 2 * NH), jnp.float32)
    z64 = jnp.zeros((NROW, FOUT), jnp.float32)

    expt, denp = _sc_exp(ss, st, sti, m16, z16)
    outt = _sc_agg(spt.reshape(NH * NP, FOUT), expt, sti, z64)
    out = _tc_merge(outt, denp)
    return out[:N].reshape(1, N, NH * FOUT)
